# baseline (speedup 1.0000x reference)
import functools

import jax
import jax.numpy as jnp
from jax.experimental import pallas as pl
from jax.experimental.pallas import tpu as pltpu


def _body(scale, nc, ctx_ref, mk_ref, mv_ref, ek_hbm, ev_hbm,
          comb_ref, eattn_hbm, mattn_ref,
          ekb_ref, evb_ref, ekstage_ref, evstage_ref, outstage_ref,
          sem_ek, sem_ev, sem_out):
    rows = ctx_ref.shape[0]
    d = ctx_ref.shape[1]
    N = ek_hbm.shape[0]
    Nc = N // nc
    first = pl.program_id(0) == 0

    nek = ekstage_ref.shape[0]
    Sek = ekstage_ref.shape[1]
    n_ek = N // Sek
    nev = evstage_ref.shape[0]
    Sev = evstage_ref.shape[1]
    n_ev = N // Sev

    def ek_copy(idx):
        return pltpu.make_async_copy(
            ek_hbm.at[pl.ds(idx * Sek, Sek), :],
            ekstage_ref.at[idx % nek], sem_ek.at[idx % nek])

    def ev_copy(idx):
        return pltpu.make_async_copy(
            ev_hbm.at[pl.ds(idx * Sev, Sev), :],
            evstage_ref.at[idx % nev], sem_ev.at[idx % nev])

    @pl.when(first)
    def _load_k():
        for idx in range(nek):
            ek_copy(idx).start()
        for idx in range(nev):
            ev_copy(idx).start()
        for idx in range(n_ek):
            ek_copy(idx).wait()
            ekb_ref[pl.ds(idx * Sek, Sek), :] = (
                ekstage_ref[idx % nek].astype(jnp.bfloat16))
            if idx + nek < n_ek:
                ek_copy(idx + nek).start()

    ctx = ctx_ref[...]
    ms = jax.lax.dot_general(
        ctx * scale, mk_ref[...], (((1,), (1,)), ((), ())),
        preferred_element_type=jnp.float32)
    ms = ms - jnp.max(ms, axis=-1, keepdims=True)
    me = jnp.exp(ms)
    m_attn = me * (1.0 / jnp.sum(me, axis=-1, keepdims=True))
    m_read = jax.lax.dot_general(
        m_attn, mv_ref[...], (((1,), (0,)), ((), ())),
        preferred_element_type=jnp.float32)
    mattn_ref[...] = m_attn
    comb_ref[:, d:] = m_read

    q = (m_read * scale).astype(jnp.bfloat16)
    maxes = []
    sums = []
    for c in range(nc):
        cols = pl.ds(c * Nc, Nc)
        es_c = jax.lax.dot_general(
            q, ekb_ref[cols, :], (((1,), (1,)), ((), ())),
            preferred_element_type=jnp.float32)
        m_c = jnp.max(es_c, axis=-1, keepdims=True)
        maxes.append(m_c)
        sums.append(jnp.sum(jnp.exp(es_c - m_c), axis=-1, keepdims=True))
    m_full = functools.reduce(jnp.maximum, maxes)
    total = sum(s * jnp.exp(m - m_full) for s, m in zip(sums, maxes))
    recip = 1.0 / total

    @pl.when(first)
    def _load_v():
        for idx in range(n_ev):
            ev_copy(idx).wait()
            evb_ref[pl.ds(idx * Sev, Sev), :] = (
                evstage_ref[idx % nev].astype(jnp.bfloat16))
            if idx + nev < n_ev:
                ev_copy(idx + nev).start()

    i = pl.program_id(0)
    acc = jnp.zeros((rows, d), dtype=jnp.float32)
    for c in range(nc):
        cols = pl.ds(c * Nc, Nc)
        es_c = jax.lax.dot_general(
            q, ekb_ref[cols, :], (((1,), (1,)), ((), ())),
            preferred_element_type=jnp.float32)
        en_c = jnp.exp(es_c - m_full) * recip
        slot = c % 2
        out_cp = pltpu.make_async_copy(
            outstage_ref.at[slot],
            eattn_hbm.at[pl.ds(i * rows, rows), pl.ds(c * Nc, Nc)],
            sem_out.at[slot])
        if c >= 2:
            prev = pltpu.make_async_copy(
                outstage_ref.at[slot],
                eattn_hbm.at[pl.ds(i * rows, rows), pl.ds((c - 2) * Nc, Nc)],
                sem_out.at[slot])
            prev.wait()
        outstage_ref[slot] = en_c
        out_cp.start()
        acc = acc + jax.lax.dot_general(
            en_c.astype(jnp.bfloat16), evb_ref[cols, :],
            (((1,), (0,)), ((), ())),
            preferred_element_type=jnp.float32)
    comb_ref[:, :d] = acc
    for c in (nc - 2, nc - 1):
        pltpu.make_async_copy(
            outstage_ref.at[c % 2],
            eattn_hbm.at[pl.ds(i * rows, rows), pl.ds(c * Nc, Nc)],
            sem_out.at[c % 2]).wait()


def kernel(context_trajectory, motif_keys, motif_values, epi_keys, epi_values):
    B, d = context_trajectory.shape
    M = motif_keys.shape[0]
    N = epi_keys.shape[0]
    scale = 1.0 / (float(d) ** 0.5)
    bB = 256
    nc = 16
    Nc = N // nc
    grid = (B // bB,)

    full = lambda i: (0, 0)
    row = lambda i: (i, 0)

    out = pl.pallas_call(
        functools.partial(_body, scale, nc),
        grid=grid,
        in_specs=[
            pl.BlockSpec((bB, d), row),
            pl.BlockSpec((M, d), full),
            pl.BlockSpec((M, d), full),
            pl.BlockSpec(memory_space=pl.ANY),
            pl.BlockSpec(memory_space=pl.ANY),
        ],
        out_specs=[
            pl.BlockSpec((bB, 2 * d), row),
            pl.BlockSpec(memory_space=pl.ANY),
            pl.BlockSpec((bB, M), row),
        ],
        out_shape=[
            jax.ShapeDtypeStruct((B, 2 * d), jnp.float32),
            jax.ShapeDtypeStruct((B, N), jnp.float32),
            jax.ShapeDtypeStruct((B, M), jnp.float32),
        ],
        scratch_shapes=[
            pltpu.VMEM((N, d), jnp.bfloat16),
            pltpu.VMEM((N, d), jnp.bfloat16),
            pltpu.VMEM((4, 512, d), jnp.float32),
            pltpu.VMEM((4, 2048, d), jnp.float32),
            pltpu.VMEM((2, bB, N // nc), jnp.float32),
            pltpu.SemaphoreType.DMA((4,)),
            pltpu.SemaphoreType.DMA((4,)),
            pltpu.SemaphoreType.DMA((2,)),
        ],
        compiler_params=pltpu.CompilerParams(
            vmem_limit_bytes=64 * 1024 * 1024,
        ),
    )(context_trajectory, motif_keys, motif_values, epi_keys, epi_values)
    return tuple(out)


# speedup vs baseline: 1.1714x; 1.1714x over previous
import functools

import jax
import jax.numpy as jnp
from jax.experimental import pallas as pl
from jax.experimental.pallas import tpu as pltpu


def _body(scale, nc, ctx_ref, mk_ref, mv_ref, ek_hbm, ev_hbm,
          comb_ref, eattn_ref, mattn_ref,
          ekb_ref, evb_ref, stage_ref, sem):
    rows = ctx_ref.shape[0]
    d = ctx_ref.shape[1]
    N = ek_hbm.shape[0]
    Nc = N // nc

    Sc = stage_ref.shape[1]
    nstage = N // Sc
    first = pl.program_id(0) == 0

    def _src(idx):
        arr = ek_hbm if idx < nstage else ev_hbm
        return arr.at[pl.ds((idx % nstage) * Sc, Sc), :]

    def _dst(idx):
        arr = ekb_ref if idx < nstage else evb_ref
        return arr.at[pl.ds((idx % nstage) * Sc, Sc), :]

    nring = stage_ref.shape[0]

    @pl.when(first)
    def _load_kv():
        for idx in range(nring):
            pltpu.make_async_copy(_src(idx), stage_ref.at[idx % nring],
                                  sem.at[idx % nring]).start()
        for idx in range(2 * nstage):
            pltpu.make_async_copy(_src(idx), stage_ref.at[idx % nring],
                                  sem.at[idx % nring]).wait()
            _dst(idx)[...] = stage_ref[idx % nring].astype(jnp.bfloat16)
            if idx + nring < 2 * nstage:
                pltpu.make_async_copy(_src(idx + nring),
                                      stage_ref.at[idx % nring],
                                      sem.at[idx % nring]).start()

    ctx = ctx_ref[...]
    ms = jax.lax.dot_general(
        ctx * scale, mk_ref[...], (((1,), (1,)), ((), ())),
        preferred_element_type=jnp.float32)
    ms = ms - jnp.max(ms, axis=-1, keepdims=True)
    me = jnp.exp(ms)
    m_attn = me * (1.0 / jnp.sum(me, axis=-1, keepdims=True))
    m_read = jax.lax.dot_general(
        m_attn, mv_ref[...], (((1,), (0,)), ((), ())),
        preferred_element_type=jnp.float32)
    mattn_ref[...] = m_attn
    comb_ref[:, d:] = m_read

    q = (m_read * scale).astype(jnp.bfloat16)
    maxes = []
    sums = []
    for c in range(nc):
        cols = pl.ds(c * Nc, Nc)
        es_c = jax.lax.dot_general(
            q, ekb_ref[cols, :], (((1,), (1,)), ((), ())),
            preferred_element_type=jnp.float32)
        m_c = jnp.max(es_c, axis=-1, keepdims=True)
        maxes.append(m_c)
        sums.append(jnp.sum(jnp.exp(es_c - m_c), axis=-1, keepdims=True))
    m_full = functools.reduce(jnp.maximum, maxes)
    total = sum(s * jnp.exp(m - m_full) for s, m in zip(sums, maxes))
    recip = 1.0 / total
    acc = jnp.zeros((rows, d), dtype=jnp.float32)
    for c in range(nc):
        cols = pl.ds(c * Nc, Nc)
        es_c = jax.lax.dot_general(
            q, ekb_ref[cols, :], (((1,), (1,)), ((), ())),
            preferred_element_type=jnp.float32)
        en_c = jnp.exp(es_c - m_full) * recip
        eattn_ref[:, cols] = en_c
        acc = acc + jax.lax.dot_general(
            en_c.astype(jnp.bfloat16), evb_ref[cols, :],
            (((1,), (0,)), ((), ())),
            preferred_element_type=jnp.float32)
    comb_ref[:, :d] = acc


def kernel(context_trajectory, motif_keys, motif_values, epi_keys, epi_values):
    B, d = context_trajectory.shape
    M = motif_keys.shape[0]
    N = epi_keys.shape[0]
    scale = 1.0 / (float(d) ** 0.5)
    bB = 256
    nc = 16
    Nc = N // nc
    grid = (B // bB,)

    full = lambda i: (0, 0)
    row = lambda i: (i, 0)

    out = pl.pallas_call(
        functools.partial(_body, scale, nc),
        grid=grid,
        in_specs=[
            pl.BlockSpec((bB, d), row),
            pl.BlockSpec((M, d), full),
            pl.BlockSpec((M, d), full),
            pl.BlockSpec(memory_space=pl.ANY),
            pl.BlockSpec(memory_space=pl.ANY),
        ],
        out_specs=[
            pl.BlockSpec((bB, 2 * d), row),
            pl.BlockSpec((bB, N), row),
            pl.BlockSpec((bB, M), row),
        ],
        out_shape=[
            jax.ShapeDtypeStruct((B, 2 * d), jnp.float32),
            jax.ShapeDtypeStruct((B, N), jnp.float32),
            jax.ShapeDtypeStruct((B, M), jnp.float32),
        ],
        scratch_shapes=[
            pltpu.VMEM((N, d), jnp.bfloat16),
            pltpu.VMEM((N, d), jnp.bfloat16),
            pltpu.VMEM((4, 512, d), jnp.float32),
            pltpu.SemaphoreType.DMA((4,)),
        ],
        compiler_params=pltpu.CompilerParams(
            vmem_limit_bytes=64 * 1024 * 1024,
        ),
    )(context_trajectory, motif_keys, motif_values, epi_keys, epi_values)
    return tuple(out)


# speedup vs baseline: 1.3230x; 1.1294x over previous
import functools

import jax
import jax.numpy as jnp
from jax.experimental import pallas as pl
from jax.experimental.pallas import tpu as pltpu


def _body(scale, nc, ctx_ref, mk_ref, mv_ref, ek_hbm, ev_hbm,
          comb_ref, eattn_ref, mattn_ref,
          ekb_ref, evb_ref, stage_ref, sem):
    rows = ctx_ref.shape[0]
    d = ctx_ref.shape[1]
    N = ek_hbm.shape[0]
    Nc = N // nc

    Sc = stage_ref.shape[1]
    nstage = N // Sc
    first = pl.program_id(0) == 0

    def _src(idx):
        arr = ek_hbm if idx < nstage else ev_hbm
        return arr.at[pl.ds((idx % nstage) * Sc, Sc), :]

    def _dst(idx):
        arr = ekb_ref if idx < nstage else evb_ref
        return arr.at[pl.ds((idx % nstage) * Sc, Sc), :]

    nring = stage_ref.shape[0]

    @pl.when(first)
    def _load_kv():
        for idx in range(nring):
            pltpu.make_async_copy(_src(idx), stage_ref.at[idx % nring],
                                  sem.at[idx % nring]).start()
        for idx in range(2 * nstage):
            pltpu.make_async_copy(_src(idx), stage_ref.at[idx % nring],
                                  sem.at[idx % nring]).wait()
            _dst(idx)[...] = stage_ref[idx % nring].astype(jnp.bfloat16)
            if idx + nring < 2 * nstage:
                pltpu.make_async_copy(_src(idx + nring),
                                      stage_ref.at[idx % nring],
                                      sem.at[idx % nring]).start()

    ctx = ctx_ref[...]
    ms = jax.lax.dot_general(
        ctx * scale, mk_ref[...], (((1,), (1,)), ((), ())),
        preferred_element_type=jnp.float32)
    ms = ms - jnp.max(ms, axis=-1, keepdims=True)
    me = jnp.exp(ms)
    m_attn = me * (1.0 / jnp.sum(me, axis=-1, keepdims=True))
    m_read = jax.lax.dot_general(
        m_attn, mv_ref[...], (((1,), (0,)), ((), ())),
        preferred_element_type=jnp.float32)
    mattn_ref[...] = m_attn
    comb_ref[:, d:] = m_read

    q = (m_read * scale).astype(jnp.bfloat16)
    maxes = []
    sums = []
    for c in range(nc):
        cols = pl.ds(c * Nc, Nc)
        es_c = jax.lax.dot_general(
            q, ekb_ref[cols, :], (((1,), (1,)), ((), ())),
            preferred_element_type=jnp.float32)
        m_c = jnp.max(es_c, axis=-1, keepdims=True)
        maxes.append(m_c)
        sums.append(jnp.sum(jnp.exp(es_c - m_c), axis=-1, keepdims=True))
    m_full = functools.reduce(jnp.maximum, maxes)
    total = sum(s * jnp.exp(m - m_full) for s, m in zip(sums, maxes))
    recip = 1.0 / total
    acc = jnp.zeros((rows, d), dtype=jnp.float32)
    for c in range(nc):
        cols = pl.ds(c * Nc, Nc)
        es_c = jax.lax.dot_general(
            q, ekb_ref[cols, :], (((1,), (1,)), ((), ())),
            preferred_element_type=jnp.float32)
        en_c = jnp.exp(es_c - m_full) * recip
        eattn_ref[:, cols] = en_c
        acc = acc + jax.lax.dot_general(
            en_c.astype(jnp.bfloat16), evb_ref[cols, :],
            (((1,), (0,)), ((), ())),
            preferred_element_type=jnp.float32)
    comb_ref[:, :d] = acc


def kernel(context_trajectory, motif_keys, motif_values, epi_keys, epi_values):
    B, d = context_trajectory.shape
    M = motif_keys.shape[0]
    N = epi_keys.shape[0]
    scale = 1.0 / (float(d) ** 0.5)
    bB = 256
    nc = 16
    Nc = N // nc
    grid = (B // bB,)

    full = lambda i: (0, 0)
    row = lambda i: (i, 0)

    out = pl.pallas_call(
        functools.partial(_body, scale, nc),
        grid=grid,
        in_specs=[
            pl.BlockSpec((bB, d), row),
            pl.BlockSpec((M, d), full),
            pl.BlockSpec((M, d), full),
            pl.BlockSpec(memory_space=pl.ANY),
            pl.BlockSpec(memory_space=pl.ANY),
        ],
        out_specs=[
            pl.BlockSpec((bB, 2 * d), row),
            pl.BlockSpec((bB, N), row),
            pl.BlockSpec((bB, M), row),
        ],
        out_shape=[
            jax.ShapeDtypeStruct((B, 2 * d), jnp.float32),
            jax.ShapeDtypeStruct((B, N), jnp.float32),
            jax.ShapeDtypeStruct((B, M), jnp.float32),
        ],
        scratch_shapes=[
            pltpu.VMEM((N, d), jnp.bfloat16),
            pltpu.VMEM((N, d), jnp.bfloat16),
            pltpu.VMEM((8, 512, d), jnp.float32),
            pltpu.SemaphoreType.DMA((8,)),
        ],
        compiler_params=pltpu.CompilerParams(
            vmem_limit_bytes=64 * 1024 * 1024,
        ),
    )(context_trajectory, motif_keys, motif_values, epi_keys, epi_values)
    return tuple(out)


# speedup vs baseline: 1.3653x; 1.0320x over previous
import functools

import jax
import jax.numpy as jnp
from jax.experimental import pallas as pl
from jax.experimental.pallas import tpu as pltpu


def _body(scale, nc, ctx_ref, mk_ref, mv_ref, ek_hbm, ev_hbm,
          comb_ref, eattn_ref, mattn_ref,
          ekb_ref, evb_ref, stage_ref, sem):
    rows = ctx_ref.shape[0]
    d = ctx_ref.shape[1]
    N = ek_hbm.shape[0]
    Nc = N // nc

    Sc = stage_ref.shape[1]
    nstage = N // Sc
    first = pl.program_id(0) == 0

    def _src(idx):
        arr = ek_hbm if idx < nstage else ev_hbm
        return arr.at[pl.ds((idx % nstage) * Sc, Sc), :]

    def _dst(idx):
        arr = ekb_ref if idx < nstage else evb_ref
        return arr.at[pl.ds((idx % nstage) * Sc, Sc), :]

    nring = stage_ref.shape[0]

    @pl.when(first)
    def _load_kv():
        for idx in range(nring):
            pltpu.make_async_copy(_src(idx), stage_ref.at[idx % nring],
                                  sem.at[idx % nring]).start()
        for idx in range(2 * nstage):
            pltpu.make_async_copy(_src(idx), stage_ref.at[idx % nring],
                                  sem.at[idx % nring]).wait()
            _dst(idx)[...] = stage_ref[idx % nring].astype(jnp.bfloat16)
            if idx + nring < 2 * nstage:
                pltpu.make_async_copy(_src(idx + nring),
                                      stage_ref.at[idx % nring],
                                      sem.at[idx % nring]).start()

    ctx = ctx_ref[...]
    ms = jax.lax.dot_general(
        ctx * scale, mk_ref[...], (((1,), (1,)), ((), ())),
        preferred_element_type=jnp.float32)
    ms = ms - jnp.max(ms, axis=-1, keepdims=True)
    me = jnp.exp(ms)
    m_attn = me * (1.0 / jnp.sum(me, axis=-1, keepdims=True))
    m_read = jax.lax.dot_general(
        m_attn, mv_ref[...], (((1,), (0,)), ((), ())),
        preferred_element_type=jnp.float32)
    mattn_ref[...] = m_attn
    comb_ref[:, d:] = m_read

    q = (m_read * scale).astype(jnp.bfloat16)
    maxes = []
    sums = []
    for c in range(nc):
        cols = pl.ds(c * Nc, Nc)
        es_c = jax.lax.dot_general(
            q, ekb_ref[cols, :], (((1,), (1,)), ((), ())),
            preferred_element_type=jnp.float32)
        m_c = jnp.max(es_c, axis=-1, keepdims=True)
        maxes.append(m_c)
        sums.append(jnp.sum(jnp.exp(es_c - m_c), axis=-1, keepdims=True))
    m_full = functools.reduce(jnp.maximum, maxes)
    total = sum(s * jnp.exp(m - m_full) for s, m in zip(sums, maxes))
    recip = 1.0 / total
    acc = jnp.zeros((rows, d), dtype=jnp.float32)
    for c in range(nc):
        cols = pl.ds(c * Nc, Nc)
        es_c = jax.lax.dot_general(
            q, ekb_ref[cols, :], (((1,), (1,)), ((), ())),
            preferred_element_type=jnp.float32)
        en_c = jnp.exp(es_c - m_full) * recip
        eattn_ref[:, cols] = en_c
        acc = acc + jax.lax.dot_general(
            en_c.astype(jnp.bfloat16), evb_ref[cols, :],
            (((1,), (0,)), ((), ())),
            preferred_element_type=jnp.float32)
    comb_ref[:, :d] = acc


def kernel(context_trajectory, motif_keys, motif_values, epi_keys, epi_values):
    B, d = context_trajectory.shape
    M = motif_keys.shape[0]
    N = epi_keys.shape[0]
    scale = 1.0 / (float(d) ** 0.5)
    bB = 256
    nc = 16
    Nc = N // nc
    grid = (B // bB,)

    full = lambda i: (0, 0)
    row = lambda i: (i, 0)

    out = pl.pallas_call(
        functools.partial(_body, scale, nc),
        grid=grid,
        in_specs=[
            pl.BlockSpec((bB, d), row),
            pl.BlockSpec((M, d), full),
            pl.BlockSpec((M, d), full),
            pl.BlockSpec(memory_space=pl.ANY),
            pl.BlockSpec(memory_space=pl.ANY),
        ],
        out_specs=[
            pl.BlockSpec((bB, 2 * d), row),
            pl.BlockSpec((bB, N), row),
            pl.BlockSpec((bB, M), row),
        ],
        out_shape=[
            jax.ShapeDtypeStruct((B, 2 * d), jnp.float32),
            jax.ShapeDtypeStruct((B, N), jnp.float32),
            jax.ShapeDtypeStruct((B, M), jnp.float32),
        ],
        scratch_shapes=[
            pltpu.VMEM((N, d), jnp.bfloat16),
            pltpu.VMEM((N, d), jnp.bfloat16),
            pltpu.VMEM((10, 512, d), jnp.float32),
            pltpu.SemaphoreType.DMA((10,)),
        ],
        compiler_params=pltpu.CompilerParams(
            vmem_limit_bytes=64 * 1024 * 1024,
        ),
    )(context_trajectory, motif_keys, motif_values, epi_keys, epi_values)
    return tuple(out)
